# baseline (speedup 1.0000x reference)
import functools

import jax
import jax.numpy as jnp
from jax import lax
from jax.experimental import pallas as pl
from jax.experimental.pallas import tpu as pltpu
from jax.experimental.pallas import tpu_sc as plsc

NC = 2
NS = 16
NW = NC * NS
L = 16
CHUNK = 128
NBUF = 8
K = 4
FP = 128


def _emb_body(table_hbm, idx_hbm, out_hbm, idx_p, idx_v, rows_v, gsem, wsem):
    wid = lax.axis_index("s") * NC + lax.axis_index("c")
    rows_per_tile = idx_hbm.shape[0] // NW
    nf = 26
    n_chunks = idx_v.shape[0] // CHUNK
    n_outer = n_chunks // NBUF

    lanes = lax.iota(jnp.int32, L)
    tail_mask = lanes < (nf - L)
    stage_rows = idx_p.shape[0]
    n_stages = rows_per_tile // stage_rows

    def compact(r, s):
        base = lanes + (s * stage_rows + r) * nf
        plsc.store_scatter(idx_v, [base], idx_p[r, pl.ds(0, L)])
        plsc.store_scatter(idx_v, [base + L], idx_p[r, pl.ds(L, L)],
                           mask=tail_mask)
        return s

    def stage(s, carry):
        pltpu.sync_copy(
            idx_hbm.at[pl.ds(wid * rows_per_tile + s * stage_rows,
                             stage_rows)], idx_p)
        lax.fori_loop(0, stage_rows, compact, s)
        return carry

    lax.fori_loop(0, n_stages, stage, 0)

    def step(j, b, first_outer, last_outer):
        pltpu.make_async_copy(
            table_hbm.at[idx_v.at[pl.ds(j * CHUNK, CHUNK)]], rows_v.at[b],
            gsem.at[b]).wait()
        base = (wid * n_chunks + j) * CHUNK
        pltpu.async_copy(rows_v.at[b], out_hbm.at[pl.ds(base, CHUNK)],
                         wsem.at[b])
        if not (last_outer and b >= NBUF - K):
            b2 = (b + K) % NBUF
            if not (first_outer and b < NBUF - K):
                pltpu.make_async_copy(
                    rows_v.at[b2], out_hbm.at[pl.ds(0, CHUNK)],
                    wsem.at[b2]).wait()
            pltpu.async_copy(
                table_hbm.at[idx_v.at[pl.ds((j + K) * CHUNK, CHUNK)]],
                rows_v.at[b2], gsem.at[b2])

    for b in range(K):
        pltpu.async_copy(table_hbm.at[idx_v.at[pl.ds(b * CHUNK, CHUNK)]],
                         rows_v.at[b], gsem.at[b])

    for b in range(NBUF):
        step(b, b, True, False)

    def outer(g, carry):
        for b in range(NBUF):
            step(g * NBUF + b, b, False, False)
        return carry

    lax.fori_loop(1, n_outer - 1, outer, 0)

    for b in range(NBUF):
        step((n_outer - 1) * NBUF + b, b, False, True)

    for b in range(NBUF):
        pltpu.make_async_copy(
            rows_v.at[b], out_hbm.at[pl.ds(0, CHUNK)], wsem.at[b]).wait()


def kernel(input, weight):
    B, F = input.shape
    D = weight.shape[1]
    total = B * F
    rows_per_tile = B // NW
    n_chunks = total // (NW * CHUNK)
    idxp = jnp.pad(input, ((0, 0), (0, FP - F)))

    mesh = plsc.VectorSubcoreMesh(core_axis_name="c", subcore_axis_name="s")
    k = functools.partial(
        pl.kernel,
        mesh=mesh,
        compiler_params=pltpu.CompilerParams(use_tc_tiling_on_sc=False,
                                             needs_layout_passes=False),
        out_type=jax.ShapeDtypeStruct((total, D), weight.dtype),
        scratch_types=[
            pltpu.VMEM((rows_per_tile // 4, FP), jnp.int32),
            pltpu.VMEM((n_chunks * CHUNK,), jnp.int32),
            pltpu.VMEM((NBUF, CHUNK, D), jnp.float32),
            pltpu.SemaphoreType.DMA((NBUF,)),
            pltpu.SemaphoreType.DMA((NBUF,)),
        ],
    )(_emb_body)
    out = k(weight, idxp)
    return out.reshape(B, F, D)


# speedup vs baseline: 1.0092x; 1.0092x over previous
import functools

import jax
import jax.numpy as jnp
from jax import lax
from jax.experimental import pallas as pl
from jax.experimental.pallas import tpu as pltpu
from jax.experimental.pallas import tpu_sc as plsc

NC = 2
NS = 16
NW = NC * NS
L = 16
CHUNK = 128
NBUF = 8
K = 4
FP = 128
NF = 26


def _fmt_body(idx_hbm, out_hbm, idx_p, idx_v):
    wid = lax.axis_index("s") * NC + lax.axis_index("c")
    rows_per_tile = idx_hbm.shape[0] // NW
    lanes = lax.iota(jnp.int32, L)
    tail_mask = lanes < (NF - L)
    stage_rows = idx_p.shape[0]
    n_stages = rows_per_tile // stage_rows

    def compact(r, s):
        base = lanes + (s * stage_rows + r) * NF
        plsc.store_scatter(idx_v, [base], idx_p[r, pl.ds(0, L)])
        plsc.store_scatter(idx_v, [base + L], idx_p[r, pl.ds(L, L)],
                           mask=tail_mask)
        return s

    def stage(s, carry):
        pltpu.sync_copy(
            idx_hbm.at[pl.ds(wid * rows_per_tile + s * stage_rows,
                             stage_rows)], idx_p)
        lax.fori_loop(0, stage_rows, compact, s)
        return carry

    lax.fori_loop(0, n_stages, stage, 0)
    pltpu.sync_copy(idx_v,
                    out_hbm.at[pl.ds(wid * idx_v.shape[0], idx_v.shape[0])])


def _emb_body(table_hbm, idx_hbm, out_hbm, idx_v, rows_v, gsem, wsem):
    wid = lax.axis_index("s") * NC + lax.axis_index("c")
    n_chunks = idx_v.shape[0] // CHUNK
    n_outer = n_chunks // NBUF
    pltpu.sync_copy(idx_hbm.at[pl.ds(wid * idx_v.shape[0], idx_v.shape[0])],
                    idx_v)

    def step(j, b, first_outer, last_outer):
        pltpu.make_async_copy(
            table_hbm.at[idx_v.at[pl.ds(j * CHUNK, CHUNK)]], rows_v.at[b],
            gsem.at[b]).wait()
        base = (wid * n_chunks + j) * CHUNK
        pltpu.async_copy(rows_v.at[b], out_hbm.at[pl.ds(base, CHUNK)],
                         wsem.at[b])
        if not (last_outer and b >= NBUF - K):
            b2 = (b + K) % NBUF
            if not (first_outer and b < NBUF - K):
                pltpu.make_async_copy(
                    rows_v.at[b2], out_hbm.at[pl.ds(0, CHUNK)],
                    wsem.at[b2]).wait()
            pltpu.async_copy(
                table_hbm.at[idx_v.at[pl.ds((j + K) * CHUNK, CHUNK)]],
                rows_v.at[b2], gsem.at[b2])

    for b in range(K):
        pltpu.async_copy(table_hbm.at[idx_v.at[pl.ds(b * CHUNK, CHUNK)]],
                         rows_v.at[b], gsem.at[b])

    for b in range(NBUF):
        step(b, b, True, False)

    def outer(g, carry):
        for b in range(NBUF):
            step(g * NBUF + b, b, False, False)
        return carry

    lax.fori_loop(1, n_outer - 1, outer, 0)

    for b in range(NBUF):
        step((n_outer - 1) * NBUF + b, b, False, True)

    for b in range(NBUF):
        pltpu.make_async_copy(
            rows_v.at[b], out_hbm.at[pl.ds(0, CHUNK)], wsem.at[b]).wait()


def kernel(input, weight):
    B, F = input.shape
    D = weight.shape[1]
    total = B * F
    rows_per_tile = B // NW
    per_tile = total // NW
    idxp = jnp.pad(input, ((0, 0), (0, FP - F)))

    mesh = plsc.VectorSubcoreMesh(core_axis_name="c", subcore_axis_name="s")

    fmt = functools.partial(
        pl.kernel,
        mesh=mesh,
        compiler_params=pltpu.CompilerParams(use_tc_tiling_on_sc=True,
                                             needs_layout_passes=False),
        out_type=jax.ShapeDtypeStruct((total,), jnp.int32),
        scratch_types=[
            pltpu.VMEM((rows_per_tile // 4, FP), jnp.int32),
            pltpu.VMEM((per_tile,), jnp.int32),
        ],
    )(_fmt_body)
    idx_flat = fmt(idxp)

    emb = functools.partial(
        pl.kernel,
        mesh=mesh,
        compiler_params=pltpu.CompilerParams(use_tc_tiling_on_sc=False,
                                             needs_layout_passes=False),
        out_type=jax.ShapeDtypeStruct((total, D), weight.dtype),
        scratch_types=[
            pltpu.VMEM((per_tile,), jnp.int32),
            pltpu.VMEM((NBUF, CHUNK, D), jnp.float32),
            pltpu.SemaphoreType.DMA((NBUF,)),
            pltpu.SemaphoreType.DMA((NBUF,)),
        ],
    )(_emb_body)
    out = emb(weight, idx_flat)
    return out.reshape(B, F, D)
